# baseline (speedup 1.0000x reference)
import functools

import jax
import jax.numpy as jnp
from jax import lax
from jax.experimental import pallas as pl
from jax.experimental.pallas import tpu as pltpu
from jax.experimental.pallas import tpu_sc as plsc

NC = 2
NS = 16
NW = NC * NS
CHUNK = 128


@functools.lru_cache(maxsize=None)
def _sc_partial_sums(n_nodes: int, d: int, n_chunks0: int, n_chunks1: int):
    n_chunks = max(n_chunks0, n_chunks1)
    acc_rows = -(-(n_nodes + 1) // (NS * 128)) * (NS * 128)
    zero_chunks_per_tile = acc_rows // NS // 128
    out_rows_per_tile = acc_rows // NS
    assert d % 16 == 0

    mesh = plsc.VectorSubcoreMesh(core_axis_name="c", subcore_axis_name="s")

    @functools.partial(
        pl.kernel,
        out_type=jax.ShapeDtypeStruct((NC, acc_rows, d), jnp.float32),
        mesh=mesh,
        scratch_types=[
            pltpu.VMEM((n_chunks, CHUNK), jnp.int32),
            pltpu.VMEM((n_chunks, CHUNK), jnp.int32),
            pltpu.VMEM((CHUNK, d), jnp.float32),
            pltpu.VMEM_SHARED((acc_rows, d), jnp.float32),
            pltpu.SemaphoreType.DMA,
        ],
    )
    def sc_kernel(x_hbm, eidx_hbm, out_hbm, col_v, row_v, gbuf, acc, sem):
        cid = lax.axis_index("c")
        sid = lax.axis_index("s")

        @pl.when(cid == 0)
        def _():
            start = sid * n_chunks0
            pltpu.sync_copy(eidx_hbm.at[1, pl.ds(start, n_chunks0)],
                            col_v.at[pl.ds(0, n_chunks0)])
            pltpu.sync_copy(eidx_hbm.at[0, pl.ds(start, n_chunks0)],
                            row_v.at[pl.ds(0, n_chunks0)])

        @pl.when(cid == 1)
        def _():
            start = NS * n_chunks0 + sid * n_chunks1
            pltpu.sync_copy(eidx_hbm.at[1, pl.ds(start, n_chunks1)],
                            col_v.at[pl.ds(0, n_chunks1)])
            pltpu.sync_copy(eidx_hbm.at[0, pl.ds(start, n_chunks1)],
                            row_v.at[pl.ds(0, n_chunks1)])

        def zero_body(i, carry):
            for j in range(d // 16):
                gbuf[i, pl.ds(j * 16, 16)] = jnp.zeros((16,), jnp.float32)
            return carry
        lax.fori_loop(0, CHUNK, zero_body, 0)
        for k in range(zero_chunks_per_tile):
            pltpu.sync_copy(
                gbuf, acc.at[pl.ds((sid * zero_chunks_per_tile + k) * 128, 128)]
            )
        plsc.subcore_barrier()

        def body(j, carry):
            pltpu.async_copy(x_hbm.at[col_v.at[j]], gbuf, sem).wait()
            pltpu.sync_copy(gbuf, acc.at[row_v.at[j]], add=True)
            return carry
        my_chunks = jnp.where(cid == 0, n_chunks0, n_chunks1)
        lax.fori_loop(0, my_chunks, body, 0)
        plsc.subcore_barrier()

        pltpu.sync_copy(
            acc.at[pl.ds(sid * out_rows_per_tile, out_rows_per_tile)],
            out_hbm.at[cid, pl.ds(sid * out_rows_per_tile, out_rows_per_tile)],
        )

    return sc_kernel


@functools.lru_cache(maxsize=None)
def _combine(n_nodes: int, d: int):
    blk = 1000
    assert n_nodes % blk == 0

    def body(x_ref, a_ref, b_ref, o_ref):
        o_ref[:, :d] = x_ref[...]
        o_ref[:, d:] = a_ref[0] + b_ref[0]

    def run(x, partial):
        return pl.pallas_call(
            body,
            grid=(n_nodes // blk,),
            in_specs=[
                pl.BlockSpec((blk, d), lambda i: (i, 0)),
                pl.BlockSpec((1, blk, d), lambda i: (0, i, 0)),
                pl.BlockSpec((1, blk, d), lambda i: (1, i, 0)),
            ],
            out_specs=pl.BlockSpec((blk, 2 * d), lambda i: (i, 0)),
            out_shape=jax.ShapeDtypeStruct((n_nodes, 2 * d), jnp.float32),
        )(x, partial, partial)

    return run


FRAC0 = 0.55


def kernel(x, edge_index):
    n_nodes, d = x.shape
    n_edges = edge_index.shape[1]
    ei = edge_index.astype(jnp.int32)

    total_chunks = -(-(-(-n_edges // (NS * CHUNK))) // 8) * 8
    n0 = max(8, round(total_chunks * FRAC0 / 8) * 8)
    n1 = total_chunks - n0
    e_pad = NS * CHUNK * total_chunks
    ei3 = ei.reshape(2, n_edges // CHUNK, CHUNK) if n_edges % CHUNK == 0 \
        else None
    if ei3 is None or e_pad != n_edges:
        pad_chunks = (e_pad - n_edges + CHUNK - 1) // CHUNK
        whole = (n_edges // CHUNK) * CHUNK
        parts = [ei[:, :whole].reshape(2, whole // CHUNK, CHUNK)]
        rem = n_edges - whole
        if rem:
            pad_piece = jnp.stack(
                [jnp.full((CHUNK - rem,), n_nodes, jnp.int32),
                 jnp.zeros((CHUNK - rem,), jnp.int32)])
            tail = jnp.concatenate([ei[:, whole:], pad_piece], axis=1)
            parts.append(tail.reshape(2, 1, CHUNK))
            pad_chunks -= 1
        if pad_chunks:
            trash = jnp.stack([
                jnp.full((pad_chunks, CHUNK), n_nodes, jnp.int32),
                jnp.zeros((pad_chunks, CHUNK), jnp.int32)])
            parts.append(trash)
        ei3 = jnp.concatenate(parts, axis=1)

    partial = _sc_partial_sums(n_nodes, d, n0, n1)(x, ei3)
    return _combine(n_nodes, d)(x, partial)


# speedup vs baseline: 1.0003x; 1.0003x over previous
import functools

import jax
import jax.numpy as jnp
from jax import lax
from jax.experimental import pallas as pl
from jax.experimental.pallas import tpu as pltpu
from jax.experimental.pallas import tpu_sc as plsc

NC = 2
NS = 16
NW = NC * NS
CHUNK = 128


@functools.lru_cache(maxsize=None)
def _sc_partial_sums(n_nodes: int, d: int, n_chunks0: int, n_chunks1: int):
    n_chunks = max(n_chunks0, n_chunks1)
    acc_rows = -(-(n_nodes + 1) // (NS * 128)) * (NS * 128)
    zero_chunks_per_tile = acc_rows // NS // 128
    out_rows_per_tile = acc_rows // NS
    assert d % 16 == 0

    mesh = plsc.VectorSubcoreMesh(core_axis_name="c", subcore_axis_name="s")

    @functools.partial(
        pl.kernel,
        out_type=jax.ShapeDtypeStruct((NC, acc_rows, d), jnp.float32),
        mesh=mesh,
        scratch_types=[
            pltpu.VMEM((n_chunks, CHUNK), jnp.int32),
            pltpu.VMEM((n_chunks, CHUNK), jnp.int32),
            pltpu.VMEM((CHUNK, d), jnp.float32),
            pltpu.VMEM_SHARED((acc_rows, d), jnp.float32),
            pltpu.SemaphoreType.DMA,
        ],
    )
    def sc_kernel(x_hbm, eidx_hbm, out_hbm, col_v, row_v, gbuf, acc, sem):
        cid = lax.axis_index("c")
        sid = lax.axis_index("s")

        @pl.when(cid == 0)
        def _():
            start = sid * n_chunks0
            pltpu.sync_copy(eidx_hbm.at[1, pl.ds(start, n_chunks0)],
                            col_v.at[pl.ds(0, n_chunks0)])
            pltpu.sync_copy(eidx_hbm.at[0, pl.ds(start, n_chunks0)],
                            row_v.at[pl.ds(0, n_chunks0)])

        @pl.when(cid == 1)
        def _():
            start = NS * n_chunks0 + sid * n_chunks1
            pltpu.sync_copy(eidx_hbm.at[1, pl.ds(start, n_chunks1)],
                            col_v.at[pl.ds(0, n_chunks1)])
            pltpu.sync_copy(eidx_hbm.at[0, pl.ds(start, n_chunks1)],
                            row_v.at[pl.ds(0, n_chunks1)])

        def zero_body(i, carry):
            for j in range(d // 16):
                gbuf[i, pl.ds(j * 16, 16)] = jnp.zeros((16,), jnp.float32)
            return carry
        lax.fori_loop(0, CHUNK, zero_body, 0)
        for k in range(zero_chunks_per_tile):
            pltpu.sync_copy(
                gbuf, acc.at[pl.ds((sid * zero_chunks_per_tile + k) * 128, 128)]
            )
        plsc.subcore_barrier()

        def body(j, carry):
            pltpu.async_copy(x_hbm.at[col_v.at[j]], gbuf, sem).wait()
            pltpu.sync_copy(gbuf, acc.at[row_v.at[j]], add=True)
            return carry
        my_chunks = jnp.where(cid == 0, n_chunks0, n_chunks1)
        lax.fori_loop(0, my_chunks, body, 0)
        plsc.subcore_barrier()

        pltpu.sync_copy(
            acc.at[pl.ds(sid * out_rows_per_tile, out_rows_per_tile)],
            out_hbm.at[cid, pl.ds(sid * out_rows_per_tile, out_rows_per_tile)],
        )

    return sc_kernel


@functools.lru_cache(maxsize=None)
def _combine(n_nodes: int, d: int):
    blk = 1000
    assert n_nodes % blk == 0

    def body(x_ref, a_ref, b_ref, o_ref):
        o_ref[:, :d] = x_ref[...]
        o_ref[:, d:] = a_ref[0] + b_ref[0]

    def run(x, partial):
        return pl.pallas_call(
            body,
            grid=(n_nodes // blk,),
            in_specs=[
                pl.BlockSpec((blk, d), lambda i: (i, 0)),
                pl.BlockSpec((1, blk, d), lambda i: (0, i, 0)),
                pl.BlockSpec((1, blk, d), lambda i: (1, i, 0)),
            ],
            out_specs=pl.BlockSpec((blk, 2 * d), lambda i: (i, 0)),
            out_shape=jax.ShapeDtypeStruct((n_nodes, 2 * d), jnp.float32),
        )(x, partial, partial)

    return run


FRAC0 = 0.55


def kernel(x, edge_index):
    n_nodes, d = x.shape
    n_edges = edge_index.shape[1]
    ei = edge_index.astype(jnp.int32)

    total_chunks = -(-(-(-n_edges // (NS * CHUNK))) // 8) * 8
    n0 = max(8, round(total_chunks * FRAC0 / 8) * 8)
    n1 = total_chunks - n0
    e_pad = NS * CHUNK * total_chunks
    ei3 = ei.reshape(2, n_edges // CHUNK, CHUNK) if n_edges % CHUNK == 0 \
        else None
    if ei3 is None or e_pad != n_edges:
        pad_chunks = (e_pad - n_edges + CHUNK - 1) // CHUNK
        whole = (n_edges // CHUNK) * CHUNK
        parts = [ei[:, :whole].reshape(2, whole // CHUNK, CHUNK)]
        rem = n_edges - whole
        if rem:
            pad_piece = jnp.stack(
                [jnp.full((CHUNK - rem,), n_nodes, jnp.int32),
                 jnp.zeros((CHUNK - rem,), jnp.int32)])
            tail = jnp.concatenate([ei[:, whole:], pad_piece], axis=1)
            parts.append(tail.reshape(2, 1, CHUNK))
            pad_chunks -= 1
        if pad_chunks:
            acc_rows = -(-(n_nodes + 1) // (NS * 128)) * (NS * 128)
            spread = n_nodes + (
                jnp.arange(pad_chunks * CHUNK, dtype=jnp.int32)
                % (acc_rows - n_nodes))
            trash = jnp.stack([
                spread.reshape(pad_chunks, CHUNK),
                jnp.zeros((pad_chunks, CHUNK), jnp.int32)])
            parts.append(trash)
        ei3 = jnp.concatenate(parts, axis=1)

    partial = _sc_partial_sums(n_nodes, d, n0, n1)(x, ei3)
    return _combine(n_nodes, d)(x, partial)


# speedup vs baseline: 2.7026x; 2.7017x over previous
import functools

import jax
import jax.numpy as jnp
from jax import lax
from jax.experimental import pallas as pl
from jax.experimental.pallas import tpu as pltpu
from jax.experimental.pallas import tpu_sc as plsc

NC = 2
NS = 16
NW = NC * NS
CHUNK = 128


@functools.lru_cache(maxsize=None)
def _sc_partial_sums(n_nodes: int, d: int, n_chunks0: int, n_chunks1: int):
    n_chunks = max(n_chunks0, n_chunks1)
    acc_rows = -(-(n_nodes + 1) // (NS * 128)) * (NS * 128)
    zero_chunks_per_tile = acc_rows // NS // 128
    out_rows_per_tile = acc_rows // NS
    assert d % 16 == 0

    mesh = plsc.VectorSubcoreMesh(core_axis_name="c", subcore_axis_name="s")

    @functools.partial(
        pl.kernel,
        out_type=jax.ShapeDtypeStruct((NC, acc_rows, d), jnp.float32),
        mesh=mesh,
        scratch_types=[
            pltpu.VMEM((n_chunks, CHUNK), jnp.int32),
            pltpu.VMEM((n_chunks, CHUNK), jnp.int32),
            pltpu.VMEM((CHUNK, d), jnp.float32),
            pltpu.VMEM_SHARED((acc_rows, d), jnp.float32),
            pltpu.SemaphoreType.DMA,
        ],
    )
    def sc_kernel(x_hbm, eidx_hbm, out_hbm, col_v, row_v, gbuf, acc, sem):
        cid = lax.axis_index("c")
        sid = lax.axis_index("s")

        @pl.when(cid == 0)
        def _():
            start = sid * n_chunks0
            pltpu.sync_copy(eidx_hbm.at[1, pl.ds(start, n_chunks0)],
                            col_v.at[pl.ds(0, n_chunks0)])
            pltpu.sync_copy(eidx_hbm.at[0, pl.ds(start, n_chunks0)],
                            row_v.at[pl.ds(0, n_chunks0)])

        @pl.when(cid == 1)
        def _():
            start = NS * n_chunks0 + sid * n_chunks1
            pltpu.sync_copy(eidx_hbm.at[1, pl.ds(start, n_chunks1)],
                            col_v.at[pl.ds(0, n_chunks1)])
            pltpu.sync_copy(eidx_hbm.at[0, pl.ds(start, n_chunks1)],
                            row_v.at[pl.ds(0, n_chunks1)])

        def zero_body(i, carry):
            for j in range(d // 16):
                gbuf[i, pl.ds(j * 16, 16)] = jnp.zeros((16,), jnp.float32)
            return carry
        lax.fori_loop(0, CHUNK, zero_body, 0)
        for k in range(zero_chunks_per_tile):
            pltpu.sync_copy(
                gbuf, acc.at[pl.ds((sid * zero_chunks_per_tile + k) * 128, 128)]
            )
        plsc.subcore_barrier()

        def body(j, carry):
            pltpu.async_copy(x_hbm.at[col_v.at[j]], gbuf, sem).wait()
            pltpu.sync_copy(gbuf, acc.at[row_v.at[j]], add=True)
            return carry
        my_chunks = jnp.where(cid == 0, n_chunks0, n_chunks1)
        lax.fori_loop(0, my_chunks, body, 0)
        plsc.subcore_barrier()

        pltpu.sync_copy(
            acc.at[pl.ds(sid * out_rows_per_tile, out_rows_per_tile)],
            out_hbm.at[cid, pl.ds(sid * out_rows_per_tile, out_rows_per_tile)],
        )

    return sc_kernel


@functools.lru_cache(maxsize=None)
def _combine(n_nodes: int, d: int):
    blk = 1000
    assert n_nodes % blk == 0

    def body(x_ref, a_ref, b_ref, o_ref):
        o_ref[:, :d] = x_ref[...]
        o_ref[:, d:] = a_ref[0] + b_ref[0]

    def run(x, partial):
        return pl.pallas_call(
            body,
            grid=(n_nodes // blk,),
            in_specs=[
                pl.BlockSpec((blk, d), lambda i: (i, 0)),
                pl.BlockSpec((1, blk, d), lambda i: (0, i, 0)),
                pl.BlockSpec((1, blk, d), lambda i: (1, i, 0)),
            ],
            out_specs=pl.BlockSpec((blk, 2 * d), lambda i: (i, 0)),
            out_shape=jax.ShapeDtypeStruct((n_nodes, 2 * d), jnp.float32),
        )(x, partial, partial)

    return run


FRAC0 = 0.50


def kernel(x, edge_index):
    n_nodes, d = x.shape
    n_edges = edge_index.shape[1]
    ei = edge_index.astype(jnp.int32)

    total_chunks = -(-(-(-n_edges // (NS * CHUNK))) // 8) * 8
    n0 = max(8, round(total_chunks * FRAC0 / 8) * 8)
    n1 = total_chunks - n0
    e_pad = NS * CHUNK * total_chunks
    ei3 = ei.reshape(2, n_edges // CHUNK, CHUNK) if n_edges % CHUNK == 0 \
        else None
    if ei3 is None or e_pad != n_edges:
        pad_chunks = (e_pad - n_edges + CHUNK - 1) // CHUNK
        whole = (n_edges // CHUNK) * CHUNK
        parts = [ei[:, :whole].reshape(2, whole // CHUNK, CHUNK)]
        rem = n_edges - whole
        if rem:
            pad_piece = jnp.stack(
                [jnp.full((CHUNK - rem,), n_nodes, jnp.int32),
                 jnp.zeros((CHUNK - rem,), jnp.int32)])
            tail = jnp.concatenate([ei[:, whole:], pad_piece], axis=1)
            parts.append(tail.reshape(2, 1, CHUNK))
            pad_chunks -= 1
        if pad_chunks:
            acc_rows = -(-(n_nodes + 1) // (NS * 128)) * (NS * 128)
            ar = jnp.arange(pad_chunks * CHUNK, dtype=jnp.int32)
            trash = jnp.stack([
                (n_nodes + ar % (acc_rows - n_nodes)).reshape(pad_chunks, CHUNK),
                (ar % n_nodes).reshape(pad_chunks, CHUNK)])
            parts.append(trash)
        ei3 = jnp.concatenate(parts, axis=1)

    partial = _sc_partial_sums(n_nodes, d, n0, n1)(x, ei3)
    return _combine(n_nodes, d)(x, partial)


# speedup vs baseline: 3.4671x; 1.2829x over previous
import functools

import jax
import jax.numpy as jnp
from jax import lax
from jax.experimental import pallas as pl
from jax.experimental.pallas import tpu as pltpu
from jax.experimental.pallas import tpu_sc as plsc

NC = 2
NS = 16
NW = NC * NS
CHUNK = 128


NSLOT = 4
NGBUF = 2


@functools.lru_cache(maxsize=None)
def _sc_partial_sums(n_nodes: int, d: int, n_chunks: int):
    acc_rows = -(-(n_nodes + 1) // (NS * 128)) * (NS * 128)
    zero_chunks_per_tile = acc_rows // NS // 128
    out_rows_per_tile = acc_rows // NS
    assert d % 16 == 0 and n_chunks % NSLOT == 0

    mesh = plsc.VectorSubcoreMesh(core_axis_name="c", subcore_axis_name="s")

    @functools.partial(
        pl.kernel,
        out_type=jax.ShapeDtypeStruct((NC, acc_rows, d), jnp.float32),
        mesh=mesh,
        scratch_types=[
            pltpu.VMEM((NSLOT, 2, CHUNK), jnp.int32),
            pltpu.VMEM((NGBUF, CHUNK, d), jnp.float32),
            pltpu.VMEM_SHARED((acc_rows, d), jnp.float32),
            [pltpu.SemaphoreType.DMA] * NSLOT,
            [pltpu.SemaphoreType.DMA] * NGBUF,
            pltpu.SemaphoreType.DMA,
        ],
    )
    def sc_kernel(x_hbm, eidx_hbm, out_hbm, islots, gbufs, acc,
                  isems, gsems, ssem):
        cid = lax.axis_index("c")
        sid = lax.axis_index("s")
        start = (cid * NS + sid) * n_chunks

        zbuf = gbufs.at[0]
        def zero_body(i, carry):
            for jj in range(d // 16):
                zbuf[i, pl.ds(jj * 16, 16)] = jnp.zeros((16,), jnp.float32)
            return carry
        lax.fori_loop(0, CHUNK, zero_body, 0)
        for k in range(zero_chunks_per_tile):
            pltpu.sync_copy(
                zbuf, acc.at[pl.ds((sid * zero_chunks_per_tile + k) * 128, 128)]
            )
        plsc.subcore_barrier()

        def idx_copy(j, s):
            return pltpu.make_async_copy(
                eidx_hbm.at[start + j], islots.at[s], isems[s])

        def gather(s, b):
            return pltpu.make_async_copy(
                x_hbm.at[islots.at[s, 1]], gbufs.at[b], gsems[b])

        for s in range(NSLOT - 1):
            idx_copy(s, s).start()
        idx_copy(0, 0).wait()
        gather(0, 0).start()
        gather(0, 0).wait()

        def body(t, carry):
            for u in range(NSLOT):
                j = t * NSLOT + u
                s, s1, s3 = u, (u + 1) % NSLOT, (u + 3) % NSLOT
                b, b1 = u % NGBUF, (u + 1) % NGBUF
                idx_copy(j + 1, s1).wait()
                gather(s1, b1).start()
                sd = pltpu.async_copy(
                    gbufs.at[b], acc.at[islots.at[s, 0]], ssem, add=True)
                idx_copy(j + 3, s3).start()
                sd.wait()
                gather(s1, b1).wait()
            return carry
        lax.fori_loop(0, n_chunks // NSLOT, body, 0)
        idx_copy(n_chunks + 1, (n_chunks + 1) % NSLOT).wait()
        idx_copy(n_chunks + 2, (n_chunks + 2) % NSLOT).wait()
        plsc.subcore_barrier()

        pltpu.sync_copy(
            acc.at[pl.ds(sid * out_rows_per_tile, out_rows_per_tile)],
            out_hbm.at[cid, pl.ds(sid * out_rows_per_tile, out_rows_per_tile)],
        )

    return sc_kernel


@functools.lru_cache(maxsize=None)
def _combine(n_nodes: int, d: int):
    blk = 1000
    assert n_nodes % blk == 0

    def body(x_ref, a_ref, b_ref, o_ref):
        o_ref[:, :d] = x_ref[...]
        o_ref[:, d:] = a_ref[0] + b_ref[0]

    def run(x, partial):
        return pl.pallas_call(
            body,
            grid=(n_nodes // blk,),
            in_specs=[
                pl.BlockSpec((blk, d), lambda i: (i, 0)),
                pl.BlockSpec((1, blk, d), lambda i: (0, i, 0)),
                pl.BlockSpec((1, blk, d), lambda i: (1, i, 0)),
            ],
            out_specs=pl.BlockSpec((blk, 2 * d), lambda i: (i, 0)),
            out_shape=jax.ShapeDtypeStruct((n_nodes, 2 * d), jnp.float32),
        )(x, partial, partial)

    return run


def kernel(x, edge_index):
    n_nodes, d = x.shape
    n_edges = edge_index.shape[1]
    ei = edge_index.astype(jnp.int32)

    real_chunks = -(-n_edges // CHUNK)
    n_chunks = -(-(-(-real_chunks // NW)) // NSLOT) * NSLOT
    rows_hbm = NW * n_chunks + 2 * NSLOT
    pad_chunks = rows_hbm - real_chunks

    whole = (n_edges // CHUNK) * CHUNK
    parts = [ei[:, :whole].reshape(2, whole // CHUNK, CHUNK)]
    rem = n_edges - whole
    if rem:
        pad_piece = jnp.stack(
            [jnp.full((CHUNK - rem,), n_nodes, jnp.int32),
             jnp.zeros((CHUNK - rem,), jnp.int32)])
        tail = jnp.concatenate([ei[:, whole:], pad_piece], axis=1)
        parts.append(tail.reshape(2, 1, CHUNK))
    if pad_chunks:
        acc_rows = -(-(n_nodes + 1) // (NS * 128)) * (NS * 128)
        ar = jnp.arange(pad_chunks * CHUNK, dtype=jnp.int32)
        trash = jnp.stack([
            (n_nodes + ar % (acc_rows - n_nodes)).reshape(pad_chunks, CHUNK),
            (ar % n_nodes).reshape(pad_chunks, CHUNK)])
        parts.append(trash)
    eidx4 = jnp.swapaxes(jnp.concatenate(parts, axis=1), 0, 1)

    partial = _sc_partial_sums(n_nodes, d, n_chunks)(x, eidx4)
    return _combine(n_nodes, d)(x, partial)


# speedup vs baseline: 3.5559x; 1.0256x over previous
import functools

import jax
import jax.numpy as jnp
from jax import lax
from jax.experimental import pallas as pl
from jax.experimental.pallas import tpu as pltpu
from jax.experimental.pallas import tpu_sc as plsc

NC = 2
NS = 16
NW = NC * NS
CHUNK = 128


NSLOT = 4
NGBUF = 2


@functools.lru_cache(maxsize=None)
def _sc_partial_sums(n_nodes: int, d: int, n_chunks: int):
    acc_rows = -(-(n_nodes + 1) // (NS * 128)) * (NS * 128)
    zero_chunks_per_tile = acc_rows // NS // 128
    out_rows_per_tile = acc_rows // NS
    assert d % 16 == 0 and n_chunks % NSLOT == 0

    mesh = plsc.VectorSubcoreMesh(core_axis_name="c", subcore_axis_name="s")

    @functools.partial(
        pl.kernel,
        out_type=jax.ShapeDtypeStruct((NC, acc_rows, d), jnp.float32),
        mesh=mesh,
        scratch_types=[
            pltpu.VMEM((NSLOT, 2, CHUNK), jnp.int32),
            pltpu.VMEM((NGBUF, CHUNK, d), jnp.float32),
            pltpu.VMEM_SHARED((acc_rows, d), jnp.float32),
            [pltpu.SemaphoreType.DMA] * NSLOT,
            [pltpu.SemaphoreType.DMA] * NGBUF,
            [pltpu.SemaphoreType.DMA] * NGBUF,
            pltpu.SemaphoreType.DMA,
        ],
    )
    def sc_kernel(x_hbm, eidx_hbm, out_hbm, islots, gbufs, acc,
                  isems, gsems, gsems2, ssem):
        cid = lax.axis_index("c")
        sid = lax.axis_index("s")
        start = (cid * NS + sid) * n_chunks

        zbuf = gbufs.at[0]
        def zero_body(i, carry):
            for jj in range(d // 16):
                zbuf[i, pl.ds(jj * 16, 16)] = jnp.zeros((16,), jnp.float32)
            return carry
        lax.fori_loop(0, CHUNK, zero_body, 0)
        for k in range(zero_chunks_per_tile):
            pltpu.sync_copy(
                zbuf, acc.at[pl.ds((sid * zero_chunks_per_tile + k) * 128, 128)]
            )
        plsc.subcore_barrier()

        def idx_copy(j, s):
            return pltpu.make_async_copy(
                eidx_hbm.at[start + j], islots.at[s], isems[s])

        H = CHUNK // 2

        def gather_a(s, b):
            return pltpu.make_async_copy(
                x_hbm.at[islots.at[s, 1, pl.ds(0, H)]],
                gbufs.at[b, pl.ds(0, H)], gsems[b])

        def gather_b(s, b):
            return pltpu.make_async_copy(
                x_hbm.at[islots.at[s, 1, pl.ds(H, H)]],
                gbufs.at[b, pl.ds(H, H)], gsems2[b])

        for s in range(NSLOT - 1):
            idx_copy(s, s).start()
        idx_copy(0, 0).wait()
        gather_a(0, 0).start()
        gather_b(0, 0).start()
        gather_a(0, 0).wait()
        gather_b(0, 0).wait()

        def body(t, carry):
            for u in range(NSLOT):
                j = t * NSLOT + u
                s, s1, s3 = u, (u + 1) % NSLOT, (u + 3) % NSLOT
                b, b1 = u % NGBUF, (u + 1) % NGBUF
                idx_copy(j + 1, s1).wait()
                gather_a(s1, b1).start()
                gather_b(s1, b1).start()
                sd = pltpu.async_copy(
                    gbufs.at[b], acc.at[islots.at[s, 0]], ssem, add=True)
                idx_copy(j + 3, s3).start()
                sd.wait()
                gather_a(s1, b1).wait()
                gather_b(s1, b1).wait()
            return carry
        lax.fori_loop(0, n_chunks // NSLOT, body, 0)
        idx_copy(n_chunks + 1, (n_chunks + 1) % NSLOT).wait()
        idx_copy(n_chunks + 2, (n_chunks + 2) % NSLOT).wait()
        plsc.subcore_barrier()

        pltpu.sync_copy(
            acc.at[pl.ds(sid * out_rows_per_tile, out_rows_per_tile)],
            out_hbm.at[cid, pl.ds(sid * out_rows_per_tile, out_rows_per_tile)],
        )

    return sc_kernel


@functools.lru_cache(maxsize=None)
def _combine(n_nodes: int, d: int):
    blk = 1000
    assert n_nodes % blk == 0

    def body(x_ref, a_ref, b_ref, o_ref):
        o_ref[:, :d] = x_ref[...]
        o_ref[:, d:] = a_ref[0] + b_ref[0]

    def run(x, partial):
        return pl.pallas_call(
            body,
            grid=(n_nodes // blk,),
            in_specs=[
                pl.BlockSpec((blk, d), lambda i: (i, 0)),
                pl.BlockSpec((1, blk, d), lambda i: (0, i, 0)),
                pl.BlockSpec((1, blk, d), lambda i: (1, i, 0)),
            ],
            out_specs=pl.BlockSpec((blk, 2 * d), lambda i: (i, 0)),
            out_shape=jax.ShapeDtypeStruct((n_nodes, 2 * d), jnp.float32),
        )(x, partial, partial)

    return run


def kernel(x, edge_index):
    n_nodes, d = x.shape
    n_edges = edge_index.shape[1]
    ei = edge_index.astype(jnp.int32)

    real_chunks = -(-n_edges // CHUNK)
    n_chunks = -(-(-(-real_chunks // NW)) // NSLOT) * NSLOT
    rows_hbm = NW * n_chunks + 2 * NSLOT
    pad_chunks = rows_hbm - real_chunks

    whole = (n_edges // CHUNK) * CHUNK
    parts = [ei[:, :whole].reshape(2, whole // CHUNK, CHUNK)]
    rem = n_edges - whole
    if rem:
        pad_piece = jnp.stack(
            [jnp.full((CHUNK - rem,), n_nodes, jnp.int32),
             jnp.zeros((CHUNK - rem,), jnp.int32)])
        tail = jnp.concatenate([ei[:, whole:], pad_piece], axis=1)
        parts.append(tail.reshape(2, 1, CHUNK))
    if pad_chunks:
        acc_rows = -(-(n_nodes + 1) // (NS * 128)) * (NS * 128)
        ar = jnp.arange(pad_chunks * CHUNK, dtype=jnp.int32)
        trash = jnp.stack([
            (n_nodes + ar % (acc_rows - n_nodes)).reshape(pad_chunks, CHUNK),
            (ar % n_nodes).reshape(pad_chunks, CHUNK)])
        parts.append(trash)
    eidx4 = jnp.swapaxes(jnp.concatenate(parts, axis=1), 0, 1)

    partial = _sc_partial_sums(n_nodes, d, n_chunks)(x, eidx4)
    return _combine(n_nodes, d)(x, partial)
